# baseline (speedup 1.0000x reference)
import functools

import jax
import jax.numpy as jnp
from jax.experimental import pallas as pl
from jax.experimental.pallas import tpu as pltpu

NB = 4
BLOCK_K = 1024
NS = 4

_DN = (((1,), (1,)), ((), ()))


def _bf16_dot(a_ref, w_ref):
    a = a_ref[...].astype(jnp.bfloat16)
    w = w_ref[...].astype(jnp.bfloat16)
    return jax.lax.dot_general(a, w, _DN, preferred_element_type=jnp.float32)


def _nnue_body(*refs, block_b, has_tail, ns):
    pov_ref = refs[0]
    w_refs = refs[1:1 + ns]
    bk_refs = refs[1 + ns:1 + 2 * ns]
    waW_refs = refs[1 + 2 * ns:1 + 3 * ns]
    baW_refs = refs[1 + 3 * ns:1 + 4 * ns]
    i = 1 + 4 * ns
    (wab_ref, bab_ref, f0W_ref, f0b_ref, f1W_ref, f1b_ref, f2W_ref,
     f2b_ref) = refs[i:i + 8]
    i += 8
    if has_tail:
        wt_ref, bt_ref, waWt_ref, baWt_ref = refs[i:i + 4]
        i += 4
    out_ref, accw_ref, accb_ref = refs[i:i + 3]

    k = pl.program_id(0)
    b = pl.program_id(1)
    nk = pl.num_programs(0)

    pw = _bf16_dot(w_refs[0], waW_refs[0])
    pb = _bf16_dot(bk_refs[0], baW_refs[0])
    for j in range(1, ns):
        pw += _bf16_dot(w_refs[j], waW_refs[j])
        pb += _bf16_dot(bk_refs[j], baW_refs[j])

    rows = pl.ds(b * block_b, block_b)

    @pl.when(k == 0)
    def _init():
        if has_tail:
            accw_ref[rows, :] = pw + _bf16_dot(wt_ref, waWt_ref)
            accb_ref[rows, :] = pb + _bf16_dot(bt_ref, baWt_ref)
        else:
            accw_ref[rows, :] = pw
            accb_ref[rows, :] = pb

    @pl.when(k > 0)
    def _accum():
        accw_ref[rows, :] += pw
        accb_ref[rows, :] += pb

    @pl.when(k == nk - 1)
    def _epilogue():
        w256 = accw_ref[rows, :] + wab_ref[...][None, :]
        b256 = accb_ref[rows, :] + bab_ref[...][None, :]
        p = pov_ref[...]
        x0 = jnp.maximum(p * w256 + (1.0 - p) * b256, 0.0)
        x1 = jnp.maximum(p * b256 + (1.0 - p) * w256, 0.0)
        f0 = f0W_ref[...]
        h = f0.shape[1] // 2
        y = (jax.lax.dot_general(x0, f0[:, :h], _DN,
                                 preferred_element_type=jnp.float32)
             + jax.lax.dot_general(x1, f0[:, h:], _DN,
                                   preferred_element_type=jnp.float32)
             + f0b_ref[...][None, :])
        y = jnp.maximum(y, 0.0)
        z = jax.lax.dot_general(y, f1W_ref[...], _DN,
                                preferred_element_type=jnp.float32)
        z = jnp.maximum(z + f1b_ref[...][None, :], 0.0)
        o = jnp.sum(z * f2W_ref[...], axis=1, keepdims=True)
        out_ref[rows, :] = o + f2b_ref[0]


def kernel(pov, white, black, wa_W, wa_b, ba_W, ba_b,
           fc0_W, fc0_b, fc1_W, fc1_b, fc2_W, fc2_b):
    B, K = white.shape
    H = wa_W.shape[0]
    block_b = B // NB
    ns = NS
    block_k = min(BLOCK_K, K // ns)
    step_k = ns * block_k
    nk = K // step_k
    k_main = nk * step_k
    tail = K - k_main

    grid = (nk, NB)
    full = lambda arr: pl.BlockSpec(arr.shape, lambda k, b: (0,) * arr.ndim)

    def act_spec(j):
        return pl.BlockSpec((block_b, block_k),
                            lambda k, b, j=j: (b, k * ns + j))

    def wt_spec(j):
        return pl.BlockSpec((H, block_k),
                            lambda k, b, j=j: (0, k * ns + j))

    in_specs = (
        [pl.BlockSpec((block_b, 1), lambda k, b: (b, 0))]
        + [act_spec(j) for j in range(ns)]
        + [act_spec(j) for j in range(ns)]
        + [wt_spec(j) for j in range(ns)]
        + [wt_spec(j) for j in range(ns)]
        + [full(wa_b), full(ba_b),
           full(fc0_W), full(fc0_b),
           full(fc1_W), full(fc1_b),
           full(fc2_W),
           pl.BlockSpec(memory_space=pltpu.SMEM)]
    )
    args = ([pov] + [white] * ns + [black] * ns + [wa_W] * ns + [ba_W] * ns
            + [wa_b, ba_b, fc0_W, fc0_b, fc1_W, fc1_b, fc2_W, fc2_b])
    if tail:
        args += [white[:, k_main:], black[:, k_main:],
                 wa_W[:, k_main:], ba_W[:, k_main:]]
        in_specs += [
            pl.BlockSpec((block_b, tail), lambda k, b: (b, 0)),
            pl.BlockSpec((block_b, tail), lambda k, b: (b, 0)),
            pl.BlockSpec((H, tail), lambda k, b: (0, 0)),
            pl.BlockSpec((H, tail), lambda k, b: (0, 0)),
        ]

    out = pl.pallas_call(
        functools.partial(_nnue_body, block_b=block_b,
                          has_tail=bool(tail), ns=ns),
        grid=grid,
        in_specs=in_specs,
        out_specs=pl.BlockSpec((B, 1), lambda k, b: (0, 0)),
        out_shape=jax.ShapeDtypeStruct((B, 1), jnp.float32),
        scratch_shapes=[
            pltpu.VMEM((B, H), jnp.float32),
            pltpu.VMEM((B, H), jnp.float32),
        ],
        compiler_params=pltpu.CompilerParams(
            dimension_semantics=("arbitrary", "parallel"),
        ),
    )(*args)
    return out


# speedup vs baseline: 3.6599x; 3.6599x over previous
import functools

import jax
import jax.numpy as jnp
from jax.experimental import pallas as pl
from jax.experimental.pallas import tpu as pltpu

NB = 4
NK = 8

_DNT = (((0,), (0,)), ((), ()))
_DN = (((1,), (1,)), ((), ()))


def _bf16_dot(a_ref, w_ref):
    a = a_ref[...].astype(jnp.bfloat16)
    w = w_ref[...].astype(jnp.bfloat16)
    return jax.lax.dot_general(a, w, _DNT, preferred_element_type=jnp.float32)


def _nnue_body(pov_ref, w_ref, bk_ref, waW_ref, wab_ref, baW_ref, bab_ref,
               f0W_ref, f0b_ref, f1W_ref, f1b_ref, f2W_ref, f2b_ref,
               out_ref, accw_ref, accb_ref, *, block_b):
    k = pl.program_id(0)
    b = pl.program_id(1)
    nk = pl.num_programs(0)

    pw = _bf16_dot(w_ref, waW_ref)
    pb = _bf16_dot(bk_ref, baW_ref)

    rows = pl.ds(b * block_b, block_b)

    @pl.when(k == 0)
    def _init():
        accw_ref[rows, :] = pw
        accb_ref[rows, :] = pb

    @pl.when(k > 0)
    def _accum():
        accw_ref[rows, :] += pw
        accb_ref[rows, :] += pb

    @pl.when(k == nk - 1)
    def _epilogue():
        w256 = accw_ref[rows, :] + wab_ref[...][None, :]
        b256 = accb_ref[rows, :] + bab_ref[...][None, :]
        p = pov_ref[...]
        x0 = jnp.maximum(p * w256 + (1.0 - p) * b256, 0.0)
        x1 = jnp.maximum(p * b256 + (1.0 - p) * w256, 0.0)
        f0 = f0W_ref[...]
        h = f0.shape[1] // 2
        y = (jax.lax.dot_general(x0, f0[:, :h], _DN,
                                 preferred_element_type=jnp.float32)
             + jax.lax.dot_general(x1, f0[:, h:], _DN,
                                   preferred_element_type=jnp.float32)
             + f0b_ref[...][None, :])
        y = jnp.maximum(y, 0.0)
        z = jax.lax.dot_general(y, f1W_ref[...], _DN,
                                preferred_element_type=jnp.float32)
        z = jnp.maximum(z + f1b_ref[...][None, :], 0.0)
        o = jnp.sum(z * f2W_ref[...], axis=1, keepdims=True)
        out_ref[rows, :] = o + f2b_ref[0]


def kernel(pov, white, black, wa_W, wa_b, ba_W, ba_b,
           fc0_W, fc0_b, fc1_W, fc1_b, fc2_W, fc2_b):
    B, K = white.shape
    H = wa_W.shape[0]
    block_b = B // NB
    block_k = K // NK

    whiteT = white.T
    blackT = black.T
    waWT = wa_W.T
    baWT = ba_W.T

    grid = (NK, NB)
    full = lambda arr: pl.BlockSpec(arr.shape, lambda k, b: (0,) * arr.ndim)

    out = pl.pallas_call(
        functools.partial(_nnue_body, block_b=block_b),
        grid=grid,
        in_specs=[
            pl.BlockSpec((block_b, 1), lambda k, b: (b, 0)),
            pl.BlockSpec((block_k, block_b), lambda k, b: (k, b)),
            pl.BlockSpec((block_k, block_b), lambda k, b: (k, b)),
            pl.BlockSpec((block_k, H), lambda k, b: (k, 0)),
            full(wa_b),
            pl.BlockSpec((block_k, H), lambda k, b: (k, 0)),
            full(ba_b),
            full(fc0_W), full(fc0_b),
            full(fc1_W), full(fc1_b),
            full(fc2_W),
            pl.BlockSpec(memory_space=pltpu.SMEM),
        ],
        out_specs=pl.BlockSpec((B, 1), lambda k, b: (0, 0)),
        out_shape=jax.ShapeDtypeStruct((B, 1), jnp.float32),
        scratch_shapes=[
            pltpu.VMEM((B, H), jnp.float32),
            pltpu.VMEM((B, H), jnp.float32),
        ],
        compiler_params=pltpu.CompilerParams(
            dimension_semantics=("arbitrary", "arbitrary"),
        ),
    )(pov, whiteT, blackT, waWT, wa_b, baWT, ba_b,
      fc0_W, fc0_b, fc1_W, fc1_b, fc2_W, fc2_b)
    return out


# speedup vs baseline: 3.8216x; 1.0442x over previous
import functools

import jax
import jax.numpy as jnp
from jax.experimental import pallas as pl
from jax.experimental.pallas import tpu as pltpu

NB = 4
NK = 8

_DNT = (((0,), (0,)), ((), ()))
_DNM = (((1,), (0,)), ((), ()))


def _bf16_dot(w_ref, a_ref):
    w = w_ref[...].astype(jnp.bfloat16)
    a = a_ref[...].astype(jnp.bfloat16)
    return jax.lax.dot_general(w, a, _DNT, preferred_element_type=jnp.float32)


def _nnue_body(pov_ref, w_ref, bk_ref, waW_ref, wab_ref, baW_ref, bab_ref,
               f0W_ref, f0b_ref, f1W_ref, f1b_ref, f2W_ref, f2b_ref,
               out_ref, accw_ref, accb_ref, *, block_b):
    k = pl.program_id(0)
    b = pl.program_id(1)
    nk = pl.num_programs(0)

    pw = _bf16_dot(waW_ref, w_ref)
    pb = _bf16_dot(baW_ref, bk_ref)

    cols = pl.ds(b * block_b, block_b)

    @pl.when(k == 0)
    def _init():
        accw_ref[:, cols] = pw
        accb_ref[:, cols] = pb

    @pl.when(k > 0)
    def _accum():
        accw_ref[:, cols] += pw
        accb_ref[:, cols] += pb

    @pl.when(k == nk - 1)
    def _epilogue():
        w256 = accw_ref[:, cols] + wab_ref[...][:, None]
        b256 = accb_ref[:, cols] + bab_ref[...][:, None]
        p = pov_ref[...][None, :]
        x0 = jnp.maximum(p * w256 + (1.0 - p) * b256, 0.0)
        x1 = jnp.maximum(p * b256 + (1.0 - p) * w256, 0.0)
        f0 = f0W_ref[...]
        h = f0.shape[1] // 2
        y = (jax.lax.dot_general(f0[:, :h], x0, _DNM,
                                 preferred_element_type=jnp.float32)
             + jax.lax.dot_general(f0[:, h:], x1, _DNM,
                                   preferred_element_type=jnp.float32)
             + f0b_ref[...][:, None])
        y = jnp.maximum(y, 0.0)
        z = jax.lax.dot_general(f1W_ref[...], y, _DNM,
                                preferred_element_type=jnp.float32)
        z = jnp.maximum(z + f1b_ref[...][:, None], 0.0)
        o = jax.lax.dot_general(f2W_ref[...], z, _DNM,
                                preferred_element_type=jnp.float32)
        out_ref[cols] = o.reshape(z.shape[1]) + f2b_ref[0]


def kernel(pov, white, black, wa_W, wa_b, ba_W, ba_b,
           fc0_W, fc0_b, fc1_W, fc1_b, fc2_W, fc2_b):
    B, K = white.shape
    H = wa_W.shape[0]
    block_b = B // NB
    block_k = K // NK

    whiteT = white.T
    blackT = black.T
    waWT = wa_W.T
    baWT = ba_W.T
    pov1 = pov.reshape(B)

    grid = (NK, NB)
    full = lambda arr: pl.BlockSpec(arr.shape, lambda k, b: (0,) * arr.ndim)

    out = pl.pallas_call(
        functools.partial(_nnue_body, block_b=block_b),
        grid=grid,
        in_specs=[
            pl.BlockSpec((block_b,), lambda k, b: (b,)),
            pl.BlockSpec((block_k, block_b), lambda k, b: (k, b)),
            pl.BlockSpec((block_k, block_b), lambda k, b: (k, b)),
            pl.BlockSpec((block_k, H), lambda k, b: (k, 0)),
            full(wa_b),
            pl.BlockSpec((block_k, H), lambda k, b: (k, 0)),
            full(ba_b),
            full(fc0_W), full(fc0_b),
            full(fc1_W), full(fc1_b),
            full(fc2_W),
            pl.BlockSpec(memory_space=pltpu.SMEM),
        ],
        out_specs=pl.BlockSpec((B,), lambda k, b: (0,)),
        out_shape=jax.ShapeDtypeStruct((B,), jnp.float32),
        scratch_shapes=[
            pltpu.VMEM((H, B), jnp.float32),
            pltpu.VMEM((H, B), jnp.float32),
        ],
        compiler_params=pltpu.CompilerParams(
            dimension_semantics=("arbitrary", "arbitrary"),
        ),
    )(pov1, whiteT, blackT, waWT, wa_b, baWT, ba_b,
      fc0_W, fc0_b, fc1_W, fc1_b, fc2_W, fc2_b)
    return out.reshape(B, 1)
